# baseline (speedup 1.0000x reference)
import functools

import jax
import jax.numpy as jnp
from jax import lax
from jax.experimental import pallas as pl
from jax.experimental.pallas import tpu as pltpu
from jax.experimental.pallas import tpu_sc as plsc

VOCAB = 512
HIDDEN = 2048
B_TOTAL = 4 * 8192

NUM_CORES = 2
NUM_SUBCORES = 16
NUM_WORKERS = NUM_CORES * NUM_SUBCORES
B_PER_W = B_TOTAL // NUM_WORKERS
K = 8
NBUF = 4
NCHUNK = B_PER_W // K

_mesh = plsc.VectorSubcoreMesh(core_axis_name="c", subcore_axis_name="s")


@functools.partial(
    pl.kernel,
    mesh=_mesh,
    out_type=jax.ShapeDtypeStruct((B_TOTAL, HIDDEN), jnp.float32),
    scratch_types=[
        pltpu.VMEM((B_PER_W,), jnp.int32),
        pltpu.VMEM((NBUF, K, HIDDEN), jnp.float32),
        pltpu.SemaphoreType.DMA,
        pltpu.SemaphoreType.DMA,
    ],
)
def _gather_kernel(table_hbm, idx_hbm, out_hbm, idx_v, rows_v, gsem, wsem):
    wid = lax.axis_index("s") * NUM_CORES + lax.axis_index("c")
    base = wid * B_PER_W

    pltpu.sync_copy(idx_hbm.at[pl.ds(base, B_PER_W)], idx_v)

    def start_gather(c, b):
        pltpu.async_copy(
            table_hbm.at[idx_v.at[pl.ds(c * K, K)]], rows_v.at[b], gsem
        )

    def wait_gather(b):
        pltpu.make_async_copy(
            table_hbm.at[idx_v.at[pl.ds(0, K)]], rows_v.at[b], gsem
        ).wait()

    def start_write(c, b):
        pltpu.async_copy(
            rows_v.at[b], out_hbm.at[pl.ds(base + c * K, K)], wsem
        )

    def wait_write(b):
        pltpu.make_async_copy(
            rows_v.at[b], out_hbm.at[pl.ds(base, K)], wsem
        ).wait()

    for b in range(NBUF):
        start_gather(b, b)

    def outer(i, _):
        c0 = i * NBUF
        for b in range(NBUF):
            c = c0 + b
            wait_gather(b)
            start_write(c, b)
            nxt = c + NBUF

            @pl.when(nxt < NCHUNK)
            def _():
                wait_write(b)
                start_gather(nxt, b)

        return 0

    lax.fori_loop(0, NCHUNK // NBUF, outer, 0)

    for b in range(NBUF):
        wait_write(b)


def kernel(input_ids, table):
    flat_ids = input_ids.reshape(B_TOTAL).astype(jnp.int32)
    out = _gather_kernel(table, flat_ids)
    return out.reshape(input_ids.shape[0], input_ids.shape[1], HIDDEN)


# speedup vs baseline: 1.3276x; 1.3276x over previous
import functools

import jax
import jax.numpy as jnp
from jax import lax
from jax.experimental import pallas as pl
from jax.experimental.pallas import tpu as pltpu
from jax.experimental.pallas import tpu_sc as plsc

VOCAB = 512
HIDDEN = 2048
B_TOTAL = 4 * 8192

NUM_CORES = 2
NUM_SUBCORES = 16
NUM_WORKERS = NUM_CORES * NUM_SUBCORES
B_PER_W = B_TOTAL // NUM_WORKERS
G = 16
NGRP = B_PER_W // G
V_PER_S = VOCAB // NUM_SUBCORES

_mesh = plsc.VectorSubcoreMesh(core_axis_name="c", subcore_axis_name="s")


@functools.partial(
    pl.kernel,
    mesh=_mesh,
    out_type=jax.ShapeDtypeStruct((B_TOTAL, HIDDEN), jnp.float32),
    scratch_types=[
        pltpu.VMEM((B_PER_W,), jnp.int32),
        pltpu.VMEM_SHARED((VOCAB, HIDDEN), jnp.float32),
        pltpu.SemaphoreType.DMA,
    ],
)
def _gather_kernel(table_hbm, idx_hbm, out_hbm, idx_v, table_sp, wsem):
    cid = lax.axis_index("c")
    sid = lax.axis_index("s")
    wid = sid * NUM_CORES + cid
    base = wid * B_PER_W

    vbase = sid * V_PER_S
    pltpu.sync_copy(
        table_hbm.at[pl.ds(vbase, V_PER_S)], table_sp.at[pl.ds(vbase, V_PER_S)]
    )
    pltpu.sync_copy(idx_hbm.at[pl.ds(base, B_PER_W)], idx_v)
    plsc.subcore_barrier()

    lane_iota = lax.iota(jnp.int32, G)

    def fire_group(g):
        rows = idx_v[pl.ds(g * G, G)]
        obase = base + g * G
        for lane in range(G):
            row = rows[lane]
            pltpu.async_copy(
                table_sp.at[pl.ds(row, 1)],
                out_hbm.at[pl.ds(obase + lane, 1)],
                wsem,
            )

    def wait_group():
        pltpu.make_async_copy(
            table_sp.at[pl.ds(0, G)], out_hbm.at[pl.ds(base, G)], wsem
        ).wait()

    fire_group(0)

    def body(g, _):
        @pl.when(g + 1 < NGRP)
        def _():
            fire_group(g + 1)

        wait_group()
        return 0

    lax.fori_loop(0, NGRP, body, 0)


def kernel(input_ids, table):
    flat_ids = input_ids.reshape(B_TOTAL).astype(jnp.int32)
    out = _gather_kernel(table, flat_ids)
    return out.reshape(input_ids.shape[0], input_ids.shape[1], HIDDEN)
